# Initial kernel scaffold; baseline (speedup 1.0000x reference)
import jax
import jax.numpy as jnp
from jax.experimental import pallas as pl


def kernel(coords, grid0, tables):
    raise NotImplementedError("write your pallas kernel here")


# speedup vs baseline: 7.4079x; 7.4079x over previous
import functools
import math

import jax
import jax.numpy as jnp
import numpy as np
from jax import lax
from jax.experimental import pallas as pl
from jax.experimental.pallas import tpu as pltpu
from jax.experimental.pallas import tpu_sc as plsc

NLEV = 16
TABLE_SIZE = 262144
TMASK = TABLE_SIZE - 1
NPTS = 262144
NOUT = 2 * NLEV

P1 = np.uint32(2654435761).view(np.int32).item()
P2 = np.uint32(805459861).view(np.int32).item()


def _level_res():
    minres = np.array([16.0, 16.0, 16.0], dtype=np.float64)
    maxres = np.array([512.0, 512.0, 512.0], dtype=np.float64)
    b = np.exp((np.log(maxres) - np.log(minres)) / (NLEV - 1))
    return [int(np.floor(minres * b**l).astype(np.int64)[0]) for l in range(NLEV)]


RES = _level_res()

NW = 32
PPW = NPTS // NW
C = 1024
NCHUNK = PPW // C
NG = C // 16

_i32 = jnp.int32
_f32 = jnp.float32


def _iota16():
    return lax.iota(_i32, 16)


def _round_half_even(u):
    t = u + 0.5
    r = t.astype(_i32)
    tie = (r.astype(_f32) == t) & ((r & 1) == 1)
    return jnp.where(tie, r - 1, r)


def _body(coords_hbm, grid_hbm, tables_hbm, out_hbm,
          grid_v, cx_v, cy_v, cz_v, idx_v, w_v, r0_v, r1_v, out_v,
          sem0, sem1):
    wid = lax.axis_index("s") * 2 + lax.axis_index("c")

    pltpu.sync_copy(grid_hbm, grid_v)

    def chunk_body(ch, _):
        base = wid * PPW + ch * C
        pltpu.sync_copy(coords_hbm.at[pl.ds(0 * NPTS + base, C)], cx_v)
        pltpu.sync_copy(coords_hbm.at[pl.ds(1 * NPTS + base, C)], cy_v)
        pltpu.sync_copy(coords_hbm.at[pl.ds(2 * NPTS + base, C)], cz_v)

        def g0(g, _):
            p = g * 16
            x = cx_v[pl.ds(p, 16)]
            y = cy_v[pl.ds(p, 16)]
            z = cz_v[pl.ds(p, 16)]
            r1f = float(RES[0] - 1)
            ix = _round_half_even((x + 1.0) * 0.5 * r1f)
            iy = _round_half_even((y + 1.0) * 0.5 * r1f)
            iz = _round_half_even((z + 1.0) * 0.5 * r1f)
            gi = (ix * (RES[0] * RES[0]) + iy * RES[0] + iz) * 2
            f0 = plsc.load_gather(grid_v, [gi])
            f1 = plsc.load_gather(grid_v, [gi + 1])
            oi = (p + _iota16()) * NOUT
            plsc.store_scatter(out_v, [oi], f0)
            plsc.store_scatter(out_v, [oi + 1], f1)
            return 0

        lax.fori_loop(0, NG, g0, 0)

        for l in range(1, NLEV):
            res = RES[l]
            rm1f = float(res - 1)
            rm1 = res - 1

            def ga(g, _, rm1f=rm1f, rm1=rm1):
                p = g * 16
                x = cx_v[pl.ds(p, 16)]
                y = cy_v[pl.ds(p, 16)]
                z = cz_v[pl.ds(p, 16)]
                ux = (x + 1.0) * 0.5 * rm1f
                uy = (y + 1.0) * 0.5 * rm1f
                uz = (z + 1.0) * 0.5 * rm1f
                fx = ux.astype(_i32)
                fy = uy.astype(_i32)
                fz = uz.astype(_i32)
                wx = ux - fx.astype(_f32)
                wy = uy - fy.astype(_f32)
                wz = uz - fz.astype(_f32)
                x1 = jnp.minimum(fx + 1, rm1)
                y1 = jnp.minimum(fy + 1, rm1)
                z1 = jnp.minimum(fz + 1, rm1)
                hx = (fx, x1)
                hy = (fy * P1, y1 * P1)
                hz = (fz * P2, z1 * P2)
                ox = (1.0 - wx, wx)
                oy = (1.0 - wy, wy)
                oz = (1.0 - wz, wz)
                for ci, (dx, dy, dz) in enumerate(
                        [(a, b, c) for a in (0, 1) for b in (0, 1) for c in (0, 1)]):
                    h = (hx[dx] ^ hy[dy] ^ hz[dz]) & TMASK
                    wc = (ox[dx] * oy[dy]) * oz[dz]
                    idx_v[pl.ds(ci * C + p, 16)] = h
                    w_v[pl.ds(ci * C + p, 16)] = wc
                return 0

            lax.fori_loop(0, NG, ga, 0)

            o0 = (2 * (l - 1)) * TABLE_SIZE
            o1 = (2 * (l - 1) + 1) * TABLE_SIZE
            d0 = pltpu.async_copy(
                tables_hbm.at[pl.ds(o0, TABLE_SIZE)].at[idx_v], r0_v, sem0)
            d1 = pltpu.async_copy(
                tables_hbm.at[pl.ds(o1, TABLE_SIZE)].at[idx_v], r1_v, sem1)
            d0.wait()
            d1.wait()

            def gc(g, _, l=l):
                p = g * 16
                acc0 = jnp.zeros((16,), _f32)
                acc1 = jnp.zeros((16,), _f32)
                for ci in range(8):
                    w = w_v[pl.ds(ci * C + p, 16)]
                    acc0 = acc0 + w * r0_v[pl.ds(ci * C + p, 16)]
                    acc1 = acc1 + w * r1_v[pl.ds(ci * C + p, 16)]
                oi = (p + _iota16()) * NOUT + (2 * l)
                plsc.store_scatter(out_v, [oi], acc0)
                plsc.store_scatter(out_v, [oi + 1], acc1)
                return 0

            lax.fori_loop(0, NG, gc, 0)

        pltpu.sync_copy(out_v, out_hbm.at[pl.ds(base * NOUT, C * NOUT)])
        return 0

    lax.fori_loop(0, NCHUNK, chunk_body, 0)


@functools.partial(
    pl.kernel,
    out_type=jax.ShapeDtypeStruct((NPTS * NOUT,), _f32),
    mesh=plsc.VectorSubcoreMesh(core_axis_name="c", subcore_axis_name="s"),
    compiler_params=pltpu.CompilerParams(needs_layout_passes=False),
    scratch_types=[
        pltpu.VMEM((RES[0] ** 3 * 2,), _f32),
        pltpu.VMEM((C,), _f32),
        pltpu.VMEM((C,), _f32),
        pltpu.VMEM((C,), _f32),
        pltpu.VMEM((8 * C,), _i32),
        pltpu.VMEM((8 * C,), _f32),
        pltpu.VMEM((8 * C,), _f32),
        pltpu.VMEM((8 * C,), _f32),
        pltpu.VMEM((C * NOUT,), _f32),
        pltpu.SemaphoreType.DMA,
        pltpu.SemaphoreType.DMA,
    ],
)
def _sc_encode(coords_hbm, grid_hbm, tables_hbm, out_hbm, *scratch):
    _body(coords_hbm, grid_hbm, tables_hbm, out_hbm, *scratch)


def kernel(coords, grid0, tables):
    coords_t = coords.T.reshape(-1)
    grid_r = grid0.transpose(1, 2, 3, 0).reshape(-1)
    tables_f = tables.reshape(-1)
    out = _sc_encode(coords_t, grid_r, tables_f)
    return out.reshape(NPTS, NOUT)
